# Initial kernel scaffold; baseline (speedup 1.0000x reference)
import jax
import jax.numpy as jnp
from jax.experimental import pallas as pl


def kernel(batch, points, edge_index, W1, b1, W2, b2, W3, b3, Wl):
    raise NotImplementedError("write your pallas kernel here")


# speedup vs baseline: 20.9767x; 20.9767x over previous
import functools

import jax
import jax.numpy as jnp
from jax import lax
from jax.experimental import pallas as pl
from jax.experimental.pallas import tpu as pltpu
from jax.experimental.pallas import tpu_sc as plsc

FP = 32


def _prep_body(x_ref, wg_ref, wd_ref, b1_ref, g_ref, d_ref):
    x = x_ref[...]
    g_ref[...] = jnp.dot(x, wg_ref[...], preferred_element_type=jnp.float32)
    d_ref[...] = (
        jnp.dot(x, wd_ref[...], preferred_element_type=jnp.float32) + b1_ref[...]
    )


def _make_mlp_body(hid):
    def _mlp_body(f_ref, w2_ref, b2_ref, h_ref):
        h1 = jnp.tanh(f_ref[...])
        h2 = jnp.tanh(
            jnp.dot(h1, w2_ref[...], preferred_element_type=jnp.float32)
            + b2_ref[...]
        )
        cols = lax.broadcasted_iota(jnp.int32, h2.shape, 1)
        h2 = jnp.where(cols == hid, 1.0, jnp.where(cols > hid, 0.0, h2))
        h_ref[...] = h2
    return _mlp_body


def _make_node_body(hid):
    def _node_body(hp_ref, y_ref, w3_ref, b3_ref, wlt_ref, t_ref, s_ref, o_ref):
        h = hp_ref[0] + hp_ref[1]
        cols = lax.broadcasted_iota(jnp.int32, h.shape, 1)
        cnt = jnp.sum(jnp.where(cols == hid, h, 0.0), axis=1, keepdims=True)
        kmat = (
            jnp.dot(h, w3_ref[...], preferred_element_type=jnp.float32)
            + cnt * b3_ref[...]
        )
        y = y_ref[...]
        yt = jnp.dot(y, t_ref[...], preferred_element_type=jnp.float32)
        s16 = jnp.dot(kmat * yt, s_ref[...], preferred_element_type=jnp.float32)
        o_ref[...] = (
            jnp.dot(y, wlt_ref[...], preferred_element_type=jnp.float32)
            + s16 / (cnt + 1.0)
        )
    return _node_body


def _make_gather(bn, p, per_w, chunk, nch):
    mesh = plsc.VectorSubcoreMesh(core_axis_name="c", subcore_axis_name="s")

    @functools.partial(
        pl.kernel,
        out_type=jax.ShapeDtypeStruct((p, FP), jnp.float32),
        mesh=mesh,
        scratch_types=[
            pltpu.VMEM((chunk,), jnp.int32),
            pltpu.VMEM((chunk,), jnp.int32),
            pltpu.VMEM((chunk, FP), jnp.float32),
            pltpu.VMEM((chunk, FP), jnp.float32),
            pltpu.SemaphoreType.DMA,
            pltpu.SemaphoreType.DMA,
        ],
        compiler_params=pltpu.CompilerParams(use_tc_tiling_on_sc=False),
    )
    def gather_k(g_hbm, d_hbm, sr_hbm, dr_hbm, out_hbm, si_v, di_v, g_v, d_v,
                 s1, s2):
        wid = lax.axis_index("s") * 2 + lax.axis_index("c")
        base = wid * per_w

        def body(i, carry):
            cb = base + i * chunk
            pltpu.sync_copy(sr_hbm.at[pl.ds(cb, chunk)], si_v)
            pltpu.sync_copy(dr_hbm.at[pl.ds(cb, chunk)], di_v)
            c1 = pltpu.async_copy(g_hbm.at[si_v], g_v, s1)
            c2 = pltpu.async_copy(d_hbm.at[di_v], d_v, s2)
            c1.wait()
            c2.wait()

            def add_row(r, c2_):
                g_v[r, pl.ds(0, 16)] = g_v[r, pl.ds(0, 16)] + d_v[r, pl.ds(0, 16)]
                g_v[r, pl.ds(16, 16)] = (
                    g_v[r, pl.ds(16, 16)] + d_v[r, pl.ds(16, 16)]
                )
                return c2_

            lax.fori_loop(0, chunk, add_row, 0)
            pltpu.sync_copy(g_v, out_hbm.at[pl.ds(cb, chunk)])
            return carry

        lax.fori_loop(0, nch, body, 0)

    return gather_k


def _make_scatter(bn, p, per_w, chunk, nch):
    mesh = plsc.VectorSubcoreMesh(core_axis_name="c", subcore_axis_name="s")
    ns = 16
    rows_per_tile = bn // ns
    zr = 125
    n_zcp = rows_per_tile // zr

    @functools.partial(
        pl.kernel,
        out_type=jax.ShapeDtypeStruct((2, bn, FP), jnp.float32),
        mesh=mesh,
        scratch_types=[
            pltpu.VMEM((chunk, FP), jnp.float32),
            pltpu.VMEM((chunk,), jnp.int32),
            pltpu.VMEM((zr, FP), jnp.float32),
            pltpu.VMEM_SHARED((bn, FP), jnp.float32),
        ],
        compiler_params=pltpu.CompilerParams(use_tc_tiling_on_sc=False),
    )
    def scatter_k(h2_hbm, dr_hbm, out_hbm, h_v, di_v, z_v, h_sh):
        cid = lax.axis_index("c")
        sid = lax.axis_index("s")
        wid = sid * 2 + cid

        def zrow(r, c):
            z_v[r, pl.ds(0, 16)] = jnp.zeros((16,), jnp.float32)
            z_v[r, pl.ds(16, 16)] = jnp.zeros((16,), jnp.float32)
            return c

        lax.fori_loop(0, zr, zrow, 0)

        def zcp(j, c):
            pltpu.sync_copy(z_v, h_sh.at[pl.ds(sid * rows_per_tile + j * zr, zr)])
            return c

        lax.fori_loop(0, n_zcp, zcp, 0)
        plsc.subcore_barrier()

        base = wid * per_w

        def body(i, c):
            cb = base + i * chunk
            pltpu.sync_copy(h2_hbm.at[pl.ds(cb, chunk)], h_v)
            pltpu.sync_copy(dr_hbm.at[pl.ds(cb, chunk)], di_v)
            pltpu.sync_copy(h_v, h_sh.at[di_v], add=True)
            return c

        lax.fori_loop(0, nch, body, 0)
        plsc.subcore_barrier()
        pltpu.sync_copy(
            h_sh.at[pl.ds(sid * rows_per_tile, rows_per_tile)],
            out_hbm.at[cid, pl.ds(sid * rows_per_tile, rows_per_tile)],
        )

    return scatter_k


def _largest_chunk(per_w):
    for c in range(128, 7, -1):
        if per_w % c == 0 and c % 8 == 0:
            return c
    raise ValueError(per_w)


def kernel(batch, points, edge_index, W1, b1, W2, b2, W3, b3, Wl):
    B, N, IN_F = batch.shape
    E = edge_index.shape[1]
    PD = points.shape[1]
    HID = W2.shape[0]
    OUT_F = Wl.shape[0]
    BN = B * N
    P = B * E
    f32 = jnp.float32

    dst = edge_index[0]
    src = edge_index[1]
    offs = (jnp.arange(B, dtype=jnp.int32) * N)[:, None]
    src_row = (src[None, :] + offs).reshape(-1)
    dst_row = (dst[None, :] + offs).reshape(-1)
    batch2 = batch.reshape(BN, IN_F)
    xin = jnp.concatenate([jnp.tile(points, (B, 1)), batch2], axis=1)

    wg = jnp.zeros((PD + IN_F, FP), f32)
    wg = wg.at[:PD, :HID].set(W1[:PD]).at[PD:, :HID].set(W1[2 * PD:2 * PD + IN_F])
    wd = jnp.zeros((PD + IN_F, FP), f32)
    wd = wd.at[:PD, :HID].set(W1[PD:2 * PD]).at[PD:, :HID].set(W1[2 * PD + IN_F:])
    b1p = jnp.zeros((1, FP), f32).at[0, :HID].set(b1)
    w2p = jnp.zeros((FP, FP), f32).at[:HID, :HID].set(W2)
    b2p = jnp.zeros((1, FP), f32).at[0, :HID].set(b2)
    w3p = jnp.zeros((FP, OUT_F * OUT_F), f32).at[:HID].set(W3)
    b3p = b3[None, :]
    wlt = Wl.T
    kk = jnp.arange(OUT_F * OUT_F)
    tmat = (kk[None, :] % OUT_F == jnp.arange(OUT_F)[:, None]).astype(f32)
    smat = (kk[:, None] // OUT_F == jnp.arange(OUT_F)[None, :]).astype(f32)

    G, D = pl.pallas_call(
        _prep_body,
        out_shape=[
            jax.ShapeDtypeStruct((BN, FP), f32),
            jax.ShapeDtypeStruct((BN, FP), f32),
        ],
    )(xin, wg, wd, b1p)

    NW = 32
    per_w = P // NW
    chunk = _largest_chunk(per_w)
    nch = per_w // chunk
    featsum = _make_gather(BN, P, per_w, chunk, nch)(G, D, src_row, dst_row)

    RB = 10000
    h2 = pl.pallas_call(
        _make_mlp_body(HID),
        grid=(P // RB,),
        in_specs=[
            pl.BlockSpec((RB, FP), lambda i: (i, 0)),
            pl.BlockSpec((FP, FP), lambda i: (0, 0)),
            pl.BlockSpec((1, FP), lambda i: (0, 0)),
        ],
        out_specs=pl.BlockSpec((RB, FP), lambda i: (i, 0)),
        out_shape=jax.ShapeDtypeStruct((P, FP), f32),
    )(featsum, w2p, b2p)

    hpart = _make_scatter(BN, P, per_w, chunk, nch)(h2, dst_row)

    NR = 2000
    out2 = pl.pallas_call(
        _make_node_body(HID),
        grid=(BN // NR,),
        in_specs=[
            pl.BlockSpec((2, NR, FP), lambda i: (0, i, 0)),
            pl.BlockSpec((NR, IN_F), lambda i: (i, 0)),
            pl.BlockSpec((FP, OUT_F * OUT_F), lambda i: (0, 0)),
            pl.BlockSpec((1, OUT_F * OUT_F), lambda i: (0, 0)),
            pl.BlockSpec((IN_F, OUT_F), lambda i: (0, 0)),
            pl.BlockSpec((IN_F, OUT_F * OUT_F), lambda i: (0, 0)),
            pl.BlockSpec((OUT_F * OUT_F, OUT_F), lambda i: (0, 0)),
        ],
        out_specs=pl.BlockSpec((NR, OUT_F), lambda i: (i, 0)),
        out_shape=jax.ShapeDtypeStruct((BN, OUT_F), f32),
    )(hpart, batch2, w3p, b3p, wlt, tmat, smat)

    return out2.reshape(B, N, OUT_F)
